# baseline (speedup 1.0000x reference)
import functools

import jax
import jax.numpy as jnp
from jax import lax
from jax.experimental import pallas as pl
from jax.experimental.pallas import tpu as pltpu
from jax.experimental.pallas import tpu_sc as plsc

N = 10000
D = 128
NSC = 2
NTILE = 16
LANES = 16
K = 128
ROWS_PER_TILE = N // NTILE


def _mm_kernel(x_ref, w_ref, o_ref, *, relu):
    x = x_ref[...]
    if relu:
        x = jnp.maximum(x, 0.0)
    o_ref[...] = jnp.dot(x, w_ref[...], preferred_element_type=jnp.float32)


def _matmul(x, w, *, relu, block_rows=2000):
    n, d = x.shape
    _, m = w.shape
    grid = (n // block_rows,)
    return pl.pallas_call(
        functools.partial(_mm_kernel, relu=relu),
        grid=grid,
        in_specs=[
            pl.BlockSpec((block_rows, d), lambda i: (i, 0)),
            pl.BlockSpec((d, m), lambda i: (0, 0)),
        ],
        out_specs=pl.BlockSpec((block_rows, m), lambda i: (i, 0)),
        out_shape=jax.ShapeDtypeStruct((n, m), jnp.float32),
    )(x, w)


def _mm2_kernel(p0_ref, p1_ref, w_ref, o_ref):
    x = jnp.maximum(p0_ref[...] + p1_ref[...], 0.0)
    o_ref[...] = jnp.dot(x, w_ref[...], preferred_element_type=jnp.float32)


def _matmul2(p0, p1, w, *, block_rows=2000):
    n, d = p0.shape
    _, m = w.shape
    grid = (n // block_rows,)
    return pl.pallas_call(
        _mm2_kernel,
        grid=grid,
        in_specs=[
            pl.BlockSpec((block_rows, d), lambda i: (i, 0)),
            pl.BlockSpec((block_rows, d), lambda i: (i, 0)),
            pl.BlockSpec((d, m), lambda i: (0, 0)),
        ],
        out_specs=pl.BlockSpec((block_rows, m), lambda i: (i, 0)),
        out_shape=jax.ShapeDtypeStruct((n, m), jnp.float32),
    )(p0, p1, w)


def _spmm_es_body(src_hbm, dst_hbm, w_hbm, table_hbm, init_hbm, out_hbm,
                  r0, r1, r2, si0, si1, si2, di0, di1, di2, wb0, wb1, wb2,
                  acc,
                  gs0, gs1, gs2, ss0, ss1, ss2, is0, is1, is2,
                  *, chunks):
    c = lax.axis_index("c")
    s = lax.axis_index("s")
    row0 = s * ROWS_PER_TILE
    bufs = [
        (r0, si0, di0, wb0, gs0, ss0, is0),
        (r1, si1, di1, wb1, gs1, ss1, is1),
        (r2, si2, di2, wb2, gs2, ss2, is2),
    ]

    def issue_idx(ch, b):
        rows, si, di, wb, gsem, ssem, isem = bufs[b]
        pltpu.async_copy(src_hbm.at[c, s, ch], si, isem)
        pltpu.async_copy(dst_hbm.at[c, s, ch], di, isem)
        pltpu.async_copy(w_hbm.at[c, s, ch], wb, isem)

    def wait_idx(ch, b):
        rows, si, di, wb, gsem, ssem, isem = bufs[b]
        pltpu.make_async_copy(src_hbm.at[c, s, ch], si, isem).wait()
        pltpu.make_async_copy(dst_hbm.at[c, s, ch], di, isem).wait()
        pltpu.make_async_copy(w_hbm.at[c, s, ch], wb, isem).wait()

    pltpu.sync_copy(
        init_hbm.at[c, pl.ds(row0, ROWS_PER_TILE)],
        acc.at[pl.ds(row0, ROWS_PER_TILE)],
    )

    issue_idx(0, 0)
    issue_idx(1, 1)
    wait_idx(0, 0)
    pltpu.async_copy(table_hbm.at[si0], r0, gs0)

    plsc.subcore_barrier()

    def trip_body(g, _):
        for b in range(3):
            ch = 3 * g + b
            rows, si, di, wb, gsem, ssem, isem = bufs[b]
            bp = (b + 2) % 3
            bn = (b + 1) % 3

            pltpu.make_async_copy(table_hbm.at[si], rows, gsem).wait()

            def scale_body(gr, _):
                for u in range(8):
                    i = gr * 8 + u
                    wv = plsc.load_gather(
                        wb, [jnp.full((LANES,), i, jnp.int32)])
                    for f in range(8):
                        rows[i, pl.ds(f * LANES, LANES)] = (
                            rows[i, pl.ds(f * LANES, LANES)] * wv)
                return 0
            lax.fori_loop(0, K // 8, scale_body, 0)

            pltpu.async_copy(rows, acc.at[di], ssem, add=True)

            @pl.when(ch >= 1)
            def _():
                rp = bufs[bp]
                pltpu.make_async_copy(rp[0], acc.at[rp[2]], rp[5]).wait()

            @pl.when(ch + 2 < chunks)
            def _():
                issue_idx(ch + 2, bp)

            @pl.when(ch + 1 < chunks)
            def _():
                wait_idx(ch + 1, bn)
                rn = bufs[bn]
                pltpu.async_copy(table_hbm.at[rn[1]], rn[0], rn[4])
        return 0

    lax.fori_loop(0, chunks // 3, trip_body, 0)

    rl = bufs[(chunks - 1) % 3]
    pltpu.make_async_copy(rl[0], acc.at[rl[2]], rl[5]).wait()

    plsc.subcore_barrier()

    pltpu.sync_copy(
        acc.at[pl.ds(row0, ROWS_PER_TILE)],
        out_hbm.at[c, pl.ds(row0, ROWS_PER_TILE)],
    )


def _spmm_es(src4, dst4, w4, table, init, *, chunks):
    mesh = plsc.VectorSubcoreMesh(core_axis_name="c", subcore_axis_name="s")
    return pl.kernel(
        functools.partial(_spmm_es_body, chunks=chunks),
        out_type=jax.ShapeDtypeStruct((2, N, 128), jnp.float32),
        mesh=mesh,
        compiler_params=pltpu.CompilerParams(use_tc_tiling_on_sc=False,
                                             needs_layout_passes=False),
        scratch_types=(
            [pltpu.VMEM((K, 128), jnp.float32)] * 3
            + [pltpu.VMEM((K,), jnp.int32)] * 3
            + [pltpu.VMEM((K,), jnp.int32)] * 3
            + [pltpu.VMEM((K,), jnp.float32)] * 3
            + [pltpu.VMEM_SHARED((N, 128), jnp.float32)]
            + [pltpu.SemaphoreType.DMA] * 9
        ),
    )(src4, dst4, w4, table, init)


def _spmm_body(src_hbm, dst_hbm, w_hbm, table_hbm, init_hbm, out_hbm,
               src_v, dst_v, w_v, r0, r1, r2, acc,
               gsem0, gsem1, gsem2, ssem0, ssem1, ssem2, *, half, chunks):
    c = lax.axis_index("c")
    s = lax.axis_index("s")
    row0 = s * ROWS_PER_TILE
    bufs = [(r0, gsem0, ssem0), (r1, gsem1, ssem1), (r2, gsem2, ssem2)]

    pltpu.sync_copy(src_hbm.at[s], src_v)
    pltpu.sync_copy(dst_hbm.at[s], dst_v)
    pltpu.sync_copy(w_hbm.at[s], w_v)

    pltpu.sync_copy(
        init_hbm.at[pl.ds(row0, ROWS_PER_TILE), pl.ds(c * half, half)],
        acc.at[pl.ds(row0, ROWS_PER_TILE)],
    )

    def adj_body(ch, _):
        for g in range(K // LANES):
            v = src_v[ch, pl.ds(g * LANES, LANES)]
            src_v[ch, pl.ds(g * LANES, LANES)] = v * 2 + c
        return 0
    lax.fori_loop(0, chunks, adj_body, 0)

    pltpu.async_copy(table_hbm.at[src_v.at[0]], r0, gsem0)
    pltpu.async_copy(table_hbm.at[src_v.at[1]], r1, gsem1)

    plsc.subcore_barrier()

    nf = half // LANES

    def trip_body(g, _):
        for b, (rows, gsem, ssem) in enumerate(bufs):
            ch = 3 * g + b
            pltpu.make_async_copy(
                table_hbm.at[src_v.at[ch]], rows, gsem).wait()

            def scale_body(gr, _):
                for u in range(8):
                    i = gr * 8 + u
                    wv = plsc.load_gather(
                        w_v,
                        [jnp.full((LANES,), ch, jnp.int32),
                         jnp.full((LANES,), i, jnp.int32)],
                    )
                    for f in range(nf):
                        rows[i, pl.ds(f * LANES, LANES)] = (
                            rows[i, pl.ds(f * LANES, LANES)] * wv)
                return 0
            lax.fori_loop(0, K // 8, scale_body, 0)

            pltpu.async_copy(rows, acc.at[dst_v.at[ch]], ssem, add=True)

            nrows, ngsem, nssem = bufs[(b + 2) % 3]
            @pl.when(ch + 2 < chunks)
            def _():
                @pl.when(ch >= 1)
                def _():
                    pltpu.make_async_copy(
                        nrows, acc.at[dst_v.at[ch]], nssem).wait()
                pltpu.async_copy(table_hbm.at[src_v.at[ch + 2]], nrows, ngsem)
        return 0

    lax.fori_loop(0, chunks // 3, trip_body, 0)

    for b in range(3):
        rows, gsem, ssem = bufs[(chunks - 3 + b) % 3]
        pltpu.make_async_copy(rows, acc.at[dst_v.at[0]], ssem).wait()

    plsc.subcore_barrier()

    pltpu.sync_copy(
        acc.at[pl.ds(row0, ROWS_PER_TILE)],
        out_hbm.at[pl.ds(row0, ROWS_PER_TILE), pl.ds(c * half, half)],
    )


def _spmm(src3, dst3, w3, table2n, init, *, half, chunks):
    mesh = plsc.VectorSubcoreMesh(core_axis_name="c", subcore_axis_name="s")
    return pl.kernel(
        functools.partial(_spmm_body, half=half, chunks=chunks),
        out_type=jax.ShapeDtypeStruct((N, 2 * half), jnp.float32),
        mesh=mesh,
        compiler_params=pltpu.CompilerParams(use_tc_tiling_on_sc=False,
                                             needs_layout_passes=False),
        scratch_types=[
            pltpu.VMEM((chunks, K), jnp.int32),
            pltpu.VMEM((chunks, K), jnp.int32),
            pltpu.VMEM((chunks, K), jnp.float32),
            pltpu.VMEM((K, half), jnp.float32),
            pltpu.VMEM((K, half), jnp.float32),
            pltpu.VMEM((K, half), jnp.float32),
            pltpu.VMEM_SHARED((N, half), jnp.float32),
            pltpu.SemaphoreType.DMA,
            pltpu.SemaphoreType.DMA,
            pltpu.SemaphoreType.DMA,
            pltpu.SemaphoreType.DMA,
            pltpu.SemaphoreType.DMA,
            pltpu.SemaphoreType.DMA,
        ],
    )(src3, dst3, w3, table2n, init)


def kernel(features, edge_index, edge_weight, W1, b1, W2, b2, W3, b3):
    e = edge_index.shape[1]

    per_w = -(-e // (2 * NTILE * 3 * K)) * 3 * K
    chunks_es = per_w // K
    epad_es = 2 * NTILE * per_w
    src_es = jnp.pad(edge_index[0], (0, epad_es - e)).reshape(
        2, NTILE, chunks_es, K)
    dst_es = jnp.pad(edge_index[1], (0, epad_es - e)).reshape(
        2, NTILE, chunks_es, K)
    w_es = jnp.pad(edge_weight, (0, epad_es - e)).reshape(
        2, NTILE, chunks_es, K)

    per_tile = -(-e // (NTILE * 3 * K)) * 3 * K
    chunks = per_tile // K
    epad = NTILE * per_tile
    src = jnp.pad(edge_index[0], (0, epad - e)).reshape(NTILE, chunks, K)
    dst = jnp.pad(edge_index[1], (0, epad - e)).reshape(NTILE, chunks, K)
    w = jnp.pad(edge_weight, (0, epad - e)).reshape(NTILE, chunks, K)

    c = W3.shape[1]
    w3p = jnp.pad(W3, ((0, 0), (0, 64 - c)))
    b3p = jnp.pad(b3, (0, 64 - c))

    zeros = jnp.zeros((N, 128), jnp.float32)
    init1 = jnp.stack([jnp.broadcast_to(b1, (N, 128)), zeros])
    init2 = jnp.stack([jnp.broadcast_to(b2, (N, 128)), zeros])
    init3 = jnp.broadcast_to(b3p, (N, 64))

    s1 = _matmul(features, W1, relu=False)
    p1 = _spmm_es(src_es, dst_es, w_es, s1, init1,
                  chunks=chunks_es)
    s2 = _matmul2(p1[0], p1[1], W2)
    p2 = _spmm_es(src_es, dst_es, w_es, s2, init2,
                  chunks=chunks_es)
    s3 = _matmul2(p2[0], p2[1], w3p)
    p3 = _spmm(src, dst, w, s3.reshape(2 * N, 32), init3, half=32,
               chunks=chunks)
    return p3[:, :c]


# speedup vs baseline: 2.7141x; 2.7141x over previous
import functools

import jax
import jax.numpy as jnp
from jax import lax
from jax.experimental import pallas as pl
from jax.experimental.pallas import tpu as pltpu
from jax.experimental.pallas import tpu_sc as plsc

N = 10000
D = 128
NSC = 2
NTILE = 16
LANES = 16
K = 128
ROWS_PER_TILE = N // NTILE


def _mm_kernel(x_ref, w_ref, o_ref, *, relu):
    x = x_ref[...]
    if relu:
        x = jnp.maximum(x, 0.0)
    o_ref[...] = jnp.dot(x, w_ref[...], preferred_element_type=jnp.float32)


def _matmul(x, w, *, relu, block_rows=2000):
    n, d = x.shape
    _, m = w.shape
    grid = (n // block_rows,)
    return pl.pallas_call(
        functools.partial(_mm_kernel, relu=relu),
        grid=grid,
        in_specs=[
            pl.BlockSpec((block_rows, d), lambda i: (i, 0)),
            pl.BlockSpec((d, m), lambda i: (0, 0)),
        ],
        out_specs=pl.BlockSpec((block_rows, m), lambda i: (i, 0)),
        out_shape=jax.ShapeDtypeStruct((n, m), jnp.float32),
    )(x, w)


def _mm2_kernel(p0_ref, p1_ref, w_ref, o_ref):
    x = jnp.maximum(p0_ref[...] + p1_ref[...], 0.0)
    o_ref[...] = jnp.dot(x, w_ref[...], preferred_element_type=jnp.float32)


def _matmul2(p0, p1, w, *, block_rows=2000):
    n, d = p0.shape
    _, m = w.shape
    grid = (n // block_rows,)
    return pl.pallas_call(
        _mm2_kernel,
        grid=grid,
        in_specs=[
            pl.BlockSpec((block_rows, d), lambda i: (i, 0)),
            pl.BlockSpec((block_rows, d), lambda i: (i, 0)),
            pl.BlockSpec((d, m), lambda i: (0, 0)),
        ],
        out_specs=pl.BlockSpec((block_rows, m), lambda i: (i, 0)),
        out_shape=jax.ShapeDtypeStruct((n, m), jnp.float32),
    )(p0, p1, w)


def _spmm_es_body(src_hbm, dst_hbm, w_hbm, table_hbm, init_hbm, out_hbm,
                  r0, r1, r2, si0, si1, si2, di0, di1, di2, wb0, wb1, wb2,
                  acc,
                  gs0, gs1, gs2, ss0, ss1, ss2, is0, is1, is2,
                  *, chunks):
    c = lax.axis_index("c")
    s = lax.axis_index("s")
    row0 = s * ROWS_PER_TILE
    bufs = [
        (r0, si0, di0, wb0, gs0, ss0, is0),
        (r1, si1, di1, wb1, gs1, ss1, is1),
        (r2, si2, di2, wb2, gs2, ss2, is2),
    ]

    def issue_idx(ch, b):
        rows, si, di, wb, gsem, ssem, isem = bufs[b]
        pltpu.async_copy(src_hbm.at[c, s, ch], si, isem)
        pltpu.async_copy(dst_hbm.at[c, s, ch], di, isem)
        pltpu.async_copy(w_hbm.at[c, s, ch], wb, isem)

    def wait_idx(ch, b):
        rows, si, di, wb, gsem, ssem, isem = bufs[b]
        pltpu.make_async_copy(src_hbm.at[c, s, ch], si, isem).wait()
        pltpu.make_async_copy(dst_hbm.at[c, s, ch], di, isem).wait()
        pltpu.make_async_copy(w_hbm.at[c, s, ch], wb, isem).wait()

    pltpu.sync_copy(
        init_hbm.at[c, pl.ds(row0, ROWS_PER_TILE)],
        acc.at[pl.ds(row0, ROWS_PER_TILE)],
    )

    issue_idx(0, 0)
    issue_idx(1, 1)
    wait_idx(0, 0)
    pltpu.async_copy(table_hbm.at[si0], r0, gs0)

    plsc.subcore_barrier()

    def trip_body(g, _):
        for b in range(3):
            ch = 3 * g + b
            rows, si, di, wb, gsem, ssem, isem = bufs[b]
            bp = (b + 2) % 3
            bn = (b + 1) % 3

            pltpu.make_async_copy(table_hbm.at[si], rows, gsem).wait()

            def scale_body(gr, _):
                for u in range(8):
                    i = gr * 8 + u
                    wv = plsc.load_gather(
                        wb, [jnp.full((LANES,), i, jnp.int32)])
                    for f in range(8):
                        rows[i, pl.ds(f * LANES, LANES)] = (
                            rows[i, pl.ds(f * LANES, LANES)] * wv)
                return 0
            lax.fori_loop(0, K // 8, scale_body, 0)

            pltpu.async_copy(rows, acc.at[di], ssem, add=True)

            @pl.when(ch >= 1)
            def _():
                rp = bufs[bp]
                pltpu.make_async_copy(rp[0], acc.at[rp[2]], rp[5]).wait()

            @pl.when(ch + 2 < chunks)
            def _():
                issue_idx(ch + 2, bp)

            @pl.when(ch + 1 < chunks)
            def _():
                wait_idx(ch + 1, bn)
                rn = bufs[bn]
                pltpu.async_copy(table_hbm.at[rn[1]], rn[0], rn[4])
        return 0

    lax.fori_loop(0, chunks // 3, trip_body, 0)

    rl = bufs[(chunks - 1) % 3]
    pltpu.make_async_copy(rl[0], acc.at[rl[2]], rl[5]).wait()

    plsc.subcore_barrier()

    pltpu.sync_copy(
        acc.at[pl.ds(row0, ROWS_PER_TILE)],
        out_hbm.at[c, pl.ds(row0, ROWS_PER_TILE)],
    )


def _spmm_es(src4, dst4, w4, table, init, *, chunks):
    mesh = plsc.VectorSubcoreMesh(core_axis_name="c", subcore_axis_name="s")
    return pl.kernel(
        functools.partial(_spmm_es_body, chunks=chunks),
        out_type=jax.ShapeDtypeStruct((2, N, 128), jnp.float32),
        mesh=mesh,
        compiler_params=pltpu.CompilerParams(use_tc_tiling_on_sc=False,
                                             needs_layout_passes=False),
        scratch_types=(
            [pltpu.VMEM((K, 128), jnp.float32)] * 3
            + [pltpu.VMEM((K,), jnp.int32)] * 3
            + [pltpu.VMEM((K,), jnp.int32)] * 3
            + [pltpu.VMEM((K,), jnp.float32)] * 3
            + [pltpu.VMEM_SHARED((N, 128), jnp.float32)]
            + [pltpu.SemaphoreType.DMA] * 9
        ),
    )(src4, dst4, w4, table, init)


def _spmm_body(src_hbm, dst_hbm, w_hbm, table_hbm, init_hbm, out_hbm,
               src_v, dst_v, w_v, r0, r1, r2, acc,
               gsem0, gsem1, gsem2, ssem0, ssem1, ssem2, *, half, chunks):
    c = lax.axis_index("c")
    s = lax.axis_index("s")
    row0 = s * ROWS_PER_TILE
    bufs = [(r0, gsem0, ssem0), (r1, gsem1, ssem1), (r2, gsem2, ssem2)]

    pltpu.sync_copy(src_hbm.at[s], src_v)
    pltpu.sync_copy(dst_hbm.at[s], dst_v)
    pltpu.sync_copy(w_hbm.at[s], w_v)

    pltpu.sync_copy(
        init_hbm.at[pl.ds(row0, ROWS_PER_TILE), pl.ds(c * half, half)],
        acc.at[pl.ds(row0, ROWS_PER_TILE)],
    )

    def adj_body(ch, _):
        for g in range(K // LANES):
            v = src_v[ch, pl.ds(g * LANES, LANES)]
            src_v[ch, pl.ds(g * LANES, LANES)] = v * 2 + c
        return 0
    lax.fori_loop(0, chunks, adj_body, 0)

    pltpu.async_copy(table_hbm.at[src_v.at[0]], r0, gsem0)
    pltpu.async_copy(table_hbm.at[src_v.at[1]], r1, gsem1)

    plsc.subcore_barrier()

    nf = half // LANES

    def trip_body(g, _):
        for b, (rows, gsem, ssem) in enumerate(bufs):
            ch = 3 * g + b
            pltpu.make_async_copy(
                table_hbm.at[src_v.at[ch]], rows, gsem).wait()

            def scale_body(gr, _):
                for u in range(8):
                    i = gr * 8 + u
                    wv = plsc.load_gather(
                        w_v,
                        [jnp.full((LANES,), ch, jnp.int32),
                         jnp.full((LANES,), i, jnp.int32)],
                    )
                    for f in range(nf):
                        rows[i, pl.ds(f * LANES, LANES)] = (
                            rows[i, pl.ds(f * LANES, LANES)] * wv)
                return 0
            lax.fori_loop(0, K // 8, scale_body, 0)

            pltpu.async_copy(rows, acc.at[dst_v.at[ch]], ssem, add=True)

            nrows, ngsem, nssem = bufs[(b + 2) % 3]
            @pl.when(ch + 2 < chunks)
            def _():
                @pl.when(ch >= 1)
                def _():
                    pltpu.make_async_copy(
                        nrows, acc.at[dst_v.at[ch]], nssem).wait()
                pltpu.async_copy(table_hbm.at[src_v.at[ch + 2]], nrows, ngsem)
        return 0

    lax.fori_loop(0, chunks // 3, trip_body, 0)

    for b in range(3):
        rows, gsem, ssem = bufs[(chunks - 3 + b) % 3]
        pltpu.make_async_copy(rows, acc.at[dst_v.at[0]], ssem).wait()

    plsc.subcore_barrier()

    pltpu.sync_copy(
        acc.at[pl.ds(row0, ROWS_PER_TILE)],
        out_hbm.at[pl.ds(row0, ROWS_PER_TILE), pl.ds(c * half, half)],
    )


def _spmm(src3, dst3, w3, table2n, init, *, half, chunks):
    mesh = plsc.VectorSubcoreMesh(core_axis_name="c", subcore_axis_name="s")
    return pl.kernel(
        functools.partial(_spmm_body, half=half, chunks=chunks),
        out_type=jax.ShapeDtypeStruct((N, 2 * half), jnp.float32),
        mesh=mesh,
        compiler_params=pltpu.CompilerParams(use_tc_tiling_on_sc=False,
                                             needs_layout_passes=False),
        scratch_types=[
            pltpu.VMEM((chunks, K), jnp.int32),
            pltpu.VMEM((chunks, K), jnp.int32),
            pltpu.VMEM((chunks, K), jnp.float32),
            pltpu.VMEM((K, half), jnp.float32),
            pltpu.VMEM((K, half), jnp.float32),
            pltpu.VMEM((K, half), jnp.float32),
            pltpu.VMEM_SHARED((N, half), jnp.float32),
            pltpu.SemaphoreType.DMA,
            pltpu.SemaphoreType.DMA,
            pltpu.SemaphoreType.DMA,
            pltpu.SemaphoreType.DMA,
            pltpu.SemaphoreType.DMA,
            pltpu.SemaphoreType.DMA,
        ],
    )(src3, dst3, w3, table2n, init)


def kernel(features, edge_index, edge_weight, W1, b1, W2, b2, W3, b3):
    e = edge_index.shape[1]

    per_w = -(-e // (2 * NTILE * 3 * K)) * 3 * K
    chunks_es = per_w // K
    epad_es = 2 * NTILE * per_w
    fill = jnp.arange(epad_es - e, dtype=jnp.int32) % N
    src_es = jnp.concatenate([edge_index[0], fill]).reshape(
        2, NTILE, chunks_es, K)
    dst_es = jnp.concatenate([edge_index[1], fill]).reshape(
        2, NTILE, chunks_es, K)
    w_es = jnp.pad(edge_weight, (0, epad_es - e)).reshape(
        2, NTILE, chunks_es, K)

    per_tile = -(-e // (NTILE * 3 * K)) * 3 * K
    chunks = per_tile // K
    epad = NTILE * per_tile
    fill16 = jnp.arange(epad - e, dtype=jnp.int32) % N
    src = jnp.concatenate([edge_index[0], fill16]).reshape(NTILE, chunks, K)
    dst = jnp.concatenate([edge_index[1], fill16]).reshape(NTILE, chunks, K)
    w = jnp.pad(edge_weight, (0, epad - e)).reshape(NTILE, chunks, K)

    c = W3.shape[1]
    w3p = jnp.pad(W3, ((0, 0), (0, 64 - c)))
    b3p = jnp.pad(b3, (0, 64 - c))

    zeros = jnp.zeros((N, 128), jnp.float32)
    init1 = jnp.stack([jnp.broadcast_to(b1, (N, 128)), zeros])
    init2 = jnp.stack([jnp.broadcast_to(b2, (N, 128)), zeros])
    init3 = jnp.broadcast_to(b3p, (N, 64))

    s1 = _matmul(features, W1, relu=False)
    p1 = _spmm_es(src_es, dst_es, w_es, s1, init1,
                  chunks=chunks_es)
    s2 = _matmul2(p1[0], p1[1], W2)
    p2 = _spmm_es(src_es, dst_es, w_es, s2, init2,
                  chunks=chunks_es)
    s3 = _matmul2(p2[0], p2[1], w3p)
    p3 = _spmm(src, dst, w, s3.reshape(2 * N, 32), init3, half=32,
               chunks=chunks)
    return p3[:, :c]
